# baseline (speedup 1.0000x reference)
import functools

import jax
import jax.numpy as jnp
from jax import lax
from jax.experimental import pallas as pl
from jax.experimental.pallas import tpu as pltpu
from jax.experimental.pallas import tpu_sc as plsc

_B = 16384
_K = 128
_L = 16
_NC = 2
_NS = 16
_NW = _NC * _NS
_BPW = _B // _NW
_CHUNK = 256
_NCHUNK = _BPW // _CHUNK
_VPR = _K // _L


def _softmax_chunk(rows):
  @pl.loop(0, _CHUNK, unroll=8)
  def _row(r):
    vals = []
    for j in range(_VPR):
      vals.append(jnp.exp(rows[r, pl.ds(j * _L, _L)]))
    part = vals[0]
    for j in range(1, _VPR):
      part = part + vals[j]
    total = plsc.cumsum(part)[_L - 1]
    total_v = jnp.zeros((_L,), jnp.float32) + total
    inv = jnp.full((_L,), 1.0, jnp.float32) / total_v
    for j in range(_VPR):
      rows[r, pl.ds(j * _L, _L)] = vals[j] * inv


def _softmax_gather_body(uid_hbm, table_hbm, out_hbm, idx_v, rows_v, gsem,
                         osem):
  wid = lax.axis_index("s") * _NC + lax.axis_index("c")
  base = wid * _BPW

  pltpu.sync_copy(uid_hbm.at[pl.ds(base, _BPW)], idx_v)

  def gather(c):
    return pltpu.async_copy(
        table_hbm.at[idx_v.at[pl.ds(c * _CHUNK, _CHUNK)]],
        rows_v.at[c % 2],
        gsem,
    )

  out_copies = [None, None]
  gat = [None, None]
  gat[0] = gather(0)
  for c in range(_NCHUNK):
    if c + 1 < _NCHUNK:
      if out_copies[(c + 1) % 2] is not None:
        for cp in out_copies[(c + 1) % 2]:
          cp.wait()
        out_copies[(c + 1) % 2] = None
      gat[(c + 1) % 2] = gather(c + 1)
    gat[c % 2].wait()
    _softmax_chunk(rows_v.at[c % 2])
    out_copies[c % 2] = [
        pltpu.async_copy(
            rows_v.at[c % 2],
            out_hbm.at[pl.ds(base + c * _CHUNK, _CHUNK)],
            osem,
        )
    ]
  for subs in out_copies:
    if subs is not None:
      for cp in subs:
        cp.wait()


@jax.jit
def _softmax_gather(uid, table):
  mesh = plsc.VectorSubcoreMesh(core_axis_name="c", subcore_axis_name="s")
  return pl.kernel(
      _softmax_gather_body,
      out_type=jax.ShapeDtypeStruct((_B, _K), jnp.float32),
      mesh=mesh,
      compiler_params=pltpu.CompilerParams(needs_layout_passes=False),
      scratch_types=[
          pltpu.VMEM((_BPW,), jnp.int32),
          pltpu.VMEM((2, _CHUNK, _K), jnp.float32),
          pltpu.SemaphoreType.DMA,
          pltpu.SemaphoreType.DMA,
      ],
  )(uid, table)


def kernel(uid, matrix):
  return _softmax_gather(uid.astype(jnp.int32), matrix)


# speedup vs baseline: 1.1585x; 1.1585x over previous
import functools

import jax
import jax.numpy as jnp
from jax import lax
from jax.experimental import pallas as pl
from jax.experimental.pallas import tpu as pltpu
from jax.experimental.pallas import tpu_sc as plsc

_B = 16384
_K = 128
_L = 16
_NC = 2
_NS = 16
_NW = _NC * _NS
_BPW = _B // _NW
_CHUNK = 256
_NCHUNK = _BPW // _CHUNK
_VPR = _K // _L


def _softmax_chunk(rows):
  @pl.loop(0, _CHUNK, unroll=4)
  def _row(r):
    vals = []
    for j in range(_VPR):
      vals.append(jnp.exp(rows[r, pl.ds(j * _L, _L)]))
    part = vals[0]
    for j in range(1, _VPR):
      part = part + vals[j]
    total = plsc.cumsum(part)[_L - 1]
    total_v = jnp.zeros((_L,), jnp.float32) + total
    inv = jnp.full((_L,), 1.0, jnp.float32) / total_v
    for j in range(_VPR):
      rows[r, pl.ds(j * _L, _L)] = vals[j] * inv


def _softmax_gather_body(uid_hbm, table_hbm, out_hbm, idx_v, rows_v, gsem0,
                         gsem1, osem0, osem1):
  gsem = [gsem0, gsem1]
  osem = [osem0, osem1]
  wid = lax.axis_index("s") * _NC + lax.axis_index("c")
  base = wid * _BPW

  pltpu.sync_copy(uid_hbm.at[pl.ds(base, _BPW)], idx_v)

  def gather(c):
    return pltpu.async_copy(
        table_hbm.at[idx_v.at[pl.ds(c * _CHUNK, _CHUNK)]],
        rows_v.at[c % 2],
        gsem[c % 2],
    )

  out_copies = [None, None]
  gat = [None, None]
  gat[0] = gather(0)
  for c in range(_NCHUNK):
    if c + 1 < _NCHUNK:
      if out_copies[(c + 1) % 2] is not None:
        for cp in out_copies[(c + 1) % 2]:
          cp.wait()
        out_copies[(c + 1) % 2] = None
      gat[(c + 1) % 2] = gather(c + 1)
    gat[c % 2].wait()
    _softmax_chunk(rows_v.at[c % 2])
    out_copies[c % 2] = [
        pltpu.async_copy(
            rows_v.at[c % 2],
            out_hbm.at[pl.ds(base + c * _CHUNK, _CHUNK)],
            osem[c % 2],
        )
    ]
  for subs in out_copies:
    if subs is not None:
      for cp in subs:
        cp.wait()


@jax.jit
def _softmax_gather(uid, table):
  mesh = plsc.VectorSubcoreMesh(core_axis_name="c", subcore_axis_name="s")
  return pl.kernel(
      _softmax_gather_body,
      out_type=jax.ShapeDtypeStruct((_B, _K), jnp.float32),
      mesh=mesh,
      compiler_params=pltpu.CompilerParams(needs_layout_passes=False),
      scratch_types=[
          pltpu.VMEM((_BPW,), jnp.int32),
          pltpu.VMEM((2, _CHUNK, _K), jnp.float32),
          pltpu.SemaphoreType.DMA,
          pltpu.SemaphoreType.DMA,
          pltpu.SemaphoreType.DMA,
          pltpu.SemaphoreType.DMA,
      ],
  )(uid, table)


def kernel(uid, matrix):
  return _softmax_gather(uid.astype(jnp.int32), matrix)


# speedup vs baseline: 1.2101x; 1.0446x over previous
import functools

import jax
import jax.numpy as jnp
from jax import lax
from jax.experimental import pallas as pl
from jax.experimental.pallas import tpu as pltpu
from jax.experimental.pallas import tpu_sc as plsc

_B = 16384
_K = 128
_L = 16
_NC = 2
_NS = 16
_NW = _NC * _NS
_BPW = _B // _NW
_CHUNK = 256
_NCHUNK = _BPW // _CHUNK
_VPR = _K // _L


def _softmax_chunk(rows):
  @pl.loop(0, _CHUNK, unroll=4)
  def _row(r):
    vals = []
    for j in range(_VPR):
      vals.append(jnp.exp(rows[r, pl.ds(j * _L, _L)]))
    part = vals[0]
    for j in range(1, _VPR):
      part = part + vals[j]
    total = plsc.cumsum(part)[_L - 1]
    total_v = jnp.zeros((_L,), jnp.float32) + total
    inv = jnp.full((_L,), 1.0, jnp.float32) / total_v
    for j in range(_VPR):
      rows[r, pl.ds(j * _L, _L)] = vals[j] * inv


def _softmax_gather_body(uid_hbm, table_hbm, out_hbm, idx_v, rows_v, gsem,
                         osem):
  wid = lax.axis_index("s") * _NC + lax.axis_index("c")
  base = wid * _BPW

  pltpu.sync_copy(uid_hbm.at[pl.ds(base, _BPW)], idx_v)

  def gather(c):
    return pltpu.async_copy(
        table_hbm.at[idx_v.at[pl.ds(c * _CHUNK, _CHUNK)]],
        rows_v.at[c % 2],
        gsem,
    )

  out_copies = [None, None]
  gat = [None, None]
  gat[0] = gather(0)
  for c in range(_NCHUNK):
    if c + 1 < _NCHUNK:
      if out_copies[(c + 1) % 2] is not None:
        for cp in out_copies[(c + 1) % 2]:
          cp.wait()
        out_copies[(c + 1) % 2] = None
      gat[(c + 1) % 2] = gather(c + 1)
    gat[c % 2].wait()
    _softmax_chunk(rows_v.at[c % 2])
    out_copies[c % 2] = [
        pltpu.async_copy(
            rows_v.at[c % 2],
            out_hbm.at[pl.ds(base + c * _CHUNK, _CHUNK)],
            osem,
        )
    ]
  for subs in out_copies:
    if subs is not None:
      for cp in subs:
        cp.wait()


@jax.jit
def _softmax_gather(uid, table):
  mesh = plsc.VectorSubcoreMesh(core_axis_name="c", subcore_axis_name="s")
  return pl.kernel(
      _softmax_gather_body,
      out_type=jax.ShapeDtypeStruct((_B, _K), jnp.float32),
      mesh=mesh,
      compiler_params=pltpu.CompilerParams(needs_layout_passes=False),
      scratch_types=[
          pltpu.VMEM((_BPW,), jnp.int32),
          pltpu.VMEM((2, _CHUNK, _K), jnp.float32),
          pltpu.SemaphoreType.DMA,
          pltpu.SemaphoreType.DMA,
      ],
  )(uid, table)


def kernel(uid, matrix):
  return _softmax_gather(uid.astype(jnp.int32), matrix)
